# Initial kernel scaffold; baseline (speedup 1.0000x reference)
import jax
import jax.numpy as jnp
from jax.experimental import pallas as pl


def kernel(edge_features, sparse_indices, kernel, bias):
    raise NotImplementedError("write your pallas kernel here")


# speedup vs baseline: 1.9029x; 1.9029x over previous
import functools

import jax
import jax.numpy as jnp
from jax import lax
from jax.experimental import pallas as pl
from jax.experimental.pallas import tpu as pltpu
from jax.experimental.pallas import tpu_sc as plsc

N_NODES = 50000
K_EDGE = 16
N_EDGES = 1600000
FILTERS = 256

_E2 = N_EDGES // 2
_C = 8000
_NCH = _E2 // _C
_NG = _C // 16
_NPAD = 51200


def _seg_body(dst_hbm, feat_hbm, out_hbm, idx_v, val_v, acc_v):
    c = lax.axis_index("c")
    s = lax.axis_index("s")

    zeros = jnp.zeros((16,), jnp.float32)

    def zbody(i, carry):
        acc_v[pl.ds(i * 16, 16)] = zeros
        return carry

    lax.fori_loop(0, N_NODES // 16, zbody, 0)

    base0 = c * _E2

    def chunk_body(ch, carry):
        base = base0 + ch * _C
        pltpu.sync_copy(dst_hbm.at[pl.ds(base, _C)], idx_v)
        pltpu.sync_copy(feat_hbm.at[pl.ds(s * N_EDGES + base, _C)], val_v)

        def g_body(g, carry2):
            idx = idx_v[pl.ds(g * 16, 16)]
            val = val_v[pl.ds(g * 16, 16)]
            plsc.addupdate_scatter(acc_v, [idx], val)
            return carry2

        lax.fori_loop(0, _NG, g_body, 0)
        return carry

    lax.fori_loop(0, _NCH, chunk_body, 0)

    pltpu.sync_copy(acc_v, out_hbm.at[pl.ds((c * K_EDGE + s) * _NPAD, N_NODES)])


_seg_sum = functools.partial(
    pl.kernel,
    out_type=jax.ShapeDtypeStruct((2 * K_EDGE * _NPAD,), jnp.float32),
    mesh=plsc.VectorSubcoreMesh(core_axis_name="c", subcore_axis_name="s"),
    scratch_types=[
        pltpu.VMEM((_C,), jnp.int32),
        pltpu.VMEM((_C,), jnp.float32),
        pltpu.VMEM((N_NODES,), jnp.float32),
    ],
    compiler_params=pltpu.CompilerParams(needs_layout_passes=False),
)(_seg_body)

_BN = 2048


def _mm_body(acc_ref, k_ref, b_ref, o_ref):
    a = acc_ref[0] + acc_ref[1]
    r = lax.dot_general(a, k_ref[...], (((0,), (0,)), ((), ())),
                        preferred_element_type=jnp.float32)
    o_ref[...] = jnp.maximum(r + b_ref[...], 0.0)


def _matmul(acc, kern, bias2d):
    return pl.pallas_call(
        _mm_body,
        grid=(_NPAD // _BN,),
        in_specs=[
            pl.BlockSpec((2, K_EDGE, _BN), lambda i: (0, 0, i)),
            pl.BlockSpec((K_EDGE, FILTERS), lambda i: (0, 0)),
            pl.BlockSpec((1, FILTERS), lambda i: (0, 0)),
        ],
        out_specs=pl.BlockSpec((_BN, FILTERS), lambda i: (i, 0)),
        out_shape=jax.ShapeDtypeStruct((_NPAD, FILTERS), jnp.float32),
    )(acc, kern, bias2d)


def kernel(edge_features, sparse_indices, kernel, bias):
    dst = sparse_indices[:, 0].astype(jnp.int32)
    acc = _seg_sum(dst, edge_features.reshape(-1))
    acc = acc.reshape(2, K_EDGE, _NPAD)
    out = _matmul(acc, kernel, bias.reshape(1, FILTERS))
    return out[:N_NODES]
